# baseline (speedup 1.0000x reference)
import functools

import jax
import jax.numpy as jnp
from jax import lax
from jax.experimental import pallas as pl
from jax.experimental.pallas import tpu as pltpu
from jax.experimental.pallas import tpu_sc as plsc

ROWS = 128
COLS = 32768
K_SEL = 32
EPS = 1e-7

_info = plsc.get_sparse_core_info()
NC = _info.num_cores
NS = _info.num_subcores
L = _info.num_lanes
NW = NC * NS
RPW = ROWS // NW
HIST_VREGS = 257
HB = HIST_VREGS * L
UNROLL = 12

_PASSES = ((31, 23, 8), (23, 15, 8), (15, 7, 8), (7, 0, 7))


def _build():
    mesh = plsc.VectorSubcoreMesh(core_axis_name="c", subcore_axis_name="s")

    @functools.partial(
        pl.kernel,
        mesh=mesh,
        out_type=jax.ShapeDtypeStruct((ROWS, COLS), jnp.float32),
        scratch_types=[
            pltpu.VMEM((COLS,), jnp.float32),
            pltpu.VMEM((COLS,), jnp.float32),
            pltpu.VMEM((2 * HIST_VREGS * L,), jnp.int32),
            pltpu.SemaphoreType.DMA,
            pltpu.SemaphoreType.DMA,
        ],
        compiler_params=pltpu.CompilerParams(needs_layout_passes=False),
    )
    def sc_topk_norm(in_hbm, out_hbm, buf_a, buf_b, hist_v, sem_a, sem_b):
        wid = lax.axis_index("s") * NC + lax.axis_index("c")
        base = wid * RPW
        lane_u = jnp.arange(L, dtype=jnp.uint32)
        ones = jnp.ones((L,), jnp.int32)

        def process_pair():
            pfx_a = jnp.uint32(0)
            pfx_b = jnp.uint32(0)
            rank_a = jnp.int32(K_SEL)
            rank_b = jnp.int32(K_SEL)

            for (m_shift, d_shift, width) in _PASSES:
                @plsc.parallel_loop(0, 2 * HB, L, unroll=4)
                def _(off):
                    hist_v[pl.ds(off, L)] = jnp.zeros((L,), jnp.int32)

                pa, pb = pfx_a, pfx_b
                idx_mask = jnp.uint32(0xFF0 if d_shift >= 4 else 0x7F0)
                first = m_shift == 31

                @plsc.parallel_loop(0, COLS, L, unroll=UNROLL)
                def _(off):
                    for buf, pfx, hoff in ((buf_a, pa, 0), (buf_b, pb, HB)):
                        v = buf[pl.ds(off, L)]
                        bits = lax.bitcast_convert_type(v, jnp.uint32)
                        if d_shift >= 4:
                            sh = (bits >> (d_shift - 4)) & idx_mask
                        else:
                            sh = (bits << (4 - d_shift)) & idx_mask
                        idx = (sh | lane_u).astype(jnp.int32)
                        if hoff:
                            idx = idx + hoff
                        if first:
                            plsc.addupdate_scatter(hist_v, [idx], ones)
                        else:
                            match = (bits >> m_shift) == pfx
                            plsc.addupdate_scatter(hist_v, [idx], ones,
                                                   mask=match)

                def suf_body(i, accs):
                    acc_a, acc_b = accs
                    d = (255 - i) * L
                    acc_a = acc_a + hist_v[pl.ds(d, L)]
                    acc_b = acc_b + hist_v[pl.ds(d + HB, L)]
                    hist_v[pl.ds(d, L)] = acc_a
                    hist_v[pl.ds(d + HB, L)] = acc_b
                    return (acc_a, acc_b)
                z = jnp.zeros((L,), jnp.int32)
                lax.fori_loop(0, 256, suf_body, (z, z))

                def bs_body(_, st):
                    lo_a, hi_a, lo_b, hi_b = st
                    mid_a = (lo_a + hi_a) // 2
                    mid_b = (lo_b + hi_b) // 2
                    c_a = jnp.sum(hist_v[pl.ds(mid_a * L, L)])
                    c_b = jnp.sum(hist_v[pl.ds(mid_b * L + HB, L)])
                    ge_a = c_a >= rank_a
                    ge_b = c_b >= rank_b
                    return (jnp.where(ge_a, mid_a, lo_a),
                            jnp.where(ge_a, hi_a, mid_a),
                            jnp.where(ge_b, mid_b, lo_b),
                            jnp.where(ge_b, hi_b, mid_b))
                i0 = jnp.int32(0)
                i256 = jnp.int32(256)
                b_a, _, b_b, _ = lax.fori_loop(0, 8, bs_body,
                                               (i0, i256, i0, i256))

                above_a = jnp.sum(hist_v[pl.ds((b_a + 1) * L, L)])
                above_b = jnp.sum(hist_v[pl.ds((b_b + 1) * L + HB, L)])
                rank_a = rank_a - above_a
                rank_b = rank_b - above_b
                w = jnp.uint32(width)
                pfx_a = (pfx_a << w) | b_a.astype(jnp.uint32)
                pfx_b = (pfx_b << w) | b_b.astype(jnp.uint32)

            delta_a = lax.bitcast_convert_type(
                jnp.broadcast_to(pfx_a, (L,)), jnp.float32) + jnp.float32(EPS)
            delta_b = lax.bitcast_convert_type(
                jnp.broadcast_to(pfx_b, (L,)), jnp.float32) + jnp.float32(EPS)

            zf = jnp.float32(0.0)
            zv = jnp.zeros((L,), jnp.float32)

            @plsc.parallel_loop(0, COLS, 4 * L, unroll=2, carry=(zv, zv))
            def accs(off, a):
                acc_a, acc_b = a
                wa = [jnp.maximum(buf_a[pl.ds(off + k * L, L)] - delta_a, zf)
                      for k in range(4)]
                wb = [jnp.maximum(buf_b[pl.ds(off + k * L, L)] - delta_b, zf)
                      for k in range(4)]
                acc_a = acc_a + ((wa[0] + wa[1]) + (wa[2] + wa[3]))
                acc_b = acc_b + ((wb[0] + wb[1]) + (wb[2] + wb[3]))
                return (acc_a, acc_b)
            s_a = jnp.sum(accs[0]) + jnp.float32(EPS)
            s_b = jnp.sum(accs[1]) + jnp.float32(EPS)
            one_v = jnp.full((L,), 1.0, jnp.float32)
            rinv_a = one_v / jnp.broadcast_to(s_a, (L,))
            rinv_b = one_v / jnp.broadcast_to(s_b, (L,))

            @plsc.parallel_loop(0, COLS, L, unroll=UNROLL)
            def _(off):
                va = buf_a[pl.ds(off, L)]
                vb = buf_b[pl.ds(off, L)]
                buf_a[pl.ds(off, L)] = jnp.maximum(va - delta_a, zf) * rinv_a
                buf_b[pl.ds(off, L)] = jnp.maximum(vb - delta_b, zf) * rinv_b

        n_pairs = RPW // 2
        ld_a = pltpu.async_copy(in_hbm.at[base], buf_a, sem_a)
        ld_b = pltpu.async_copy(in_hbm.at[base + 1], buf_b, sem_b)
        for p in range(n_pairs):
            r = base + 2 * p
            ld_a.wait()
            ld_b.wait()
            process_pair()
            st_a = pltpu.async_copy(buf_a, out_hbm.at[r], sem_a)
            st_b = pltpu.async_copy(buf_b, out_hbm.at[r + 1], sem_b)
            if p + 1 < n_pairs:
                st_a.wait()
                ld_a = pltpu.async_copy(in_hbm.at[r + 2], buf_a, sem_a)
                st_b.wait()
                ld_b = pltpu.async_copy(in_hbm.at[r + 3], buf_b, sem_b)
        st_a.wait()
        st_b.wait()

    return sc_topk_norm


_sc_kernel = _build()


def kernel(attn_s):
    return _sc_kernel(attn_s)


# speedup vs baseline: 1.0274x; 1.0274x over previous
import functools

import jax
import jax.numpy as jnp
from jax import lax
from jax.experimental import pallas as pl
from jax.experimental.pallas import tpu as pltpu
from jax.experimental.pallas import tpu_sc as plsc

ROWS = 128
COLS = 32768
K_SEL = 32
EPS = 1e-7

_info = plsc.get_sparse_core_info()
NC = _info.num_cores
NS = _info.num_subcores
L = _info.num_lanes
NW = NC * NS
RPW = ROWS // NW
HIST_VREGS = 257
HB = HIST_VREGS * L
UNROLL = 4

_PASSES = ((31, 23, 8), (23, 15, 8), (15, 7, 8), (7, 0, 7))


def _build():
    mesh = plsc.VectorSubcoreMesh(core_axis_name="c", subcore_axis_name="s")

    @functools.partial(
        pl.kernel,
        mesh=mesh,
        out_type=jax.ShapeDtypeStruct((ROWS, COLS), jnp.float32),
        scratch_types=[
            pltpu.VMEM((COLS,), jnp.float32),
            pltpu.VMEM((COLS,), jnp.float32),
            pltpu.VMEM((2 * HIST_VREGS * L,), jnp.int32),
            pltpu.SemaphoreType.DMA,
            pltpu.SemaphoreType.DMA,
        ],
        compiler_params=pltpu.CompilerParams(needs_layout_passes=False),
    )
    def sc_topk_norm(in_hbm, out_hbm, buf_a, buf_b, hist_v, sem_a, sem_b):
        wid = lax.axis_index("s") * NC + lax.axis_index("c")
        base = wid * RPW
        lane_u = jnp.arange(L, dtype=jnp.uint32)
        ones = jnp.ones((L,), jnp.int32)

        def process_pair():
            pfx_a = jnp.uint32(0)
            pfx_b = jnp.uint32(0)
            rank_a = jnp.int32(K_SEL)
            rank_b = jnp.int32(K_SEL)

            for (m_shift, d_shift, width) in _PASSES:
                @plsc.parallel_loop(0, 2 * HB, L, unroll=4)
                def _(off):
                    hist_v[pl.ds(off, L)] = jnp.zeros((L,), jnp.int32)

                pa, pb = pfx_a, pfx_b
                idx_mask = jnp.uint32(0xFF0 if d_shift >= 4 else 0x7F0)
                first = m_shift == 31

                @plsc.parallel_loop(0, COLS, L, unroll=UNROLL)
                def _(off):
                    for buf, pfx, hoff in ((buf_a, pa, 0), (buf_b, pb, HB)):
                        v = buf[pl.ds(off, L)]
                        bits = lax.bitcast_convert_type(v, jnp.uint32)
                        if d_shift >= 4:
                            sh = (bits >> (d_shift - 4)) & idx_mask
                        else:
                            sh = (bits << (4 - d_shift)) & idx_mask
                        idx = (sh | lane_u).astype(jnp.int32)
                        if hoff:
                            idx = idx + hoff
                        if first:
                            plsc.addupdate_scatter(hist_v, [idx], ones)
                        else:
                            match = (bits >> m_shift) == pfx
                            plsc.addupdate_scatter(hist_v, [idx], ones,
                                                   mask=match)

                def suf_body(i, accs):
                    acc_a, acc_b = accs
                    d = (255 - i) * L
                    acc_a = acc_a + hist_v[pl.ds(d, L)]
                    acc_b = acc_b + hist_v[pl.ds(d + HB, L)]
                    hist_v[pl.ds(d, L)] = acc_a
                    hist_v[pl.ds(d + HB, L)] = acc_b
                    return (acc_a, acc_b)
                z = jnp.zeros((L,), jnp.int32)
                lax.fori_loop(0, 256, suf_body, (z, z))

                def bs_body(_, st):
                    lo_a, hi_a, lo_b, hi_b = st
                    mid_a = (lo_a + hi_a) // 2
                    mid_b = (lo_b + hi_b) // 2
                    c_a = jnp.sum(hist_v[pl.ds(mid_a * L, L)])
                    c_b = jnp.sum(hist_v[pl.ds(mid_b * L + HB, L)])
                    ge_a = c_a >= rank_a
                    ge_b = c_b >= rank_b
                    return (jnp.where(ge_a, mid_a, lo_a),
                            jnp.where(ge_a, hi_a, mid_a),
                            jnp.where(ge_b, mid_b, lo_b),
                            jnp.where(ge_b, hi_b, mid_b))
                i0 = jnp.int32(0)
                i256 = jnp.int32(256)
                b_a, _, b_b, _ = lax.fori_loop(0, 8, bs_body,
                                               (i0, i256, i0, i256))

                above_a = jnp.sum(hist_v[pl.ds((b_a + 1) * L, L)])
                above_b = jnp.sum(hist_v[pl.ds((b_b + 1) * L + HB, L)])
                rank_a = rank_a - above_a
                rank_b = rank_b - above_b
                w = jnp.uint32(width)
                pfx_a = (pfx_a << w) | b_a.astype(jnp.uint32)
                pfx_b = (pfx_b << w) | b_b.astype(jnp.uint32)

            delta_a = lax.bitcast_convert_type(
                jnp.broadcast_to(pfx_a, (L,)), jnp.float32) + jnp.float32(EPS)
            delta_b = lax.bitcast_convert_type(
                jnp.broadcast_to(pfx_b, (L,)), jnp.float32) + jnp.float32(EPS)

            zf = jnp.float32(0.0)
            zv = jnp.zeros((L,), jnp.float32)

            @plsc.parallel_loop(0, COLS, 4 * L, unroll=2, carry=(zv, zv))
            def accs(off, a):
                acc_a, acc_b = a
                wa = [jnp.maximum(buf_a[pl.ds(off + k * L, L)] - delta_a, zf)
                      for k in range(4)]
                wb = [jnp.maximum(buf_b[pl.ds(off + k * L, L)] - delta_b, zf)
                      for k in range(4)]
                acc_a = acc_a + ((wa[0] + wa[1]) + (wa[2] + wa[3]))
                acc_b = acc_b + ((wb[0] + wb[1]) + (wb[2] + wb[3]))
                return (acc_a, acc_b)
            s_a = jnp.sum(accs[0]) + jnp.float32(EPS)
            s_b = jnp.sum(accs[1]) + jnp.float32(EPS)
            one_v = jnp.full((L,), 1.0, jnp.float32)
            rinv_a = one_v / jnp.broadcast_to(s_a, (L,))
            rinv_b = one_v / jnp.broadcast_to(s_b, (L,))

            @plsc.parallel_loop(0, COLS, L, unroll=UNROLL)
            def _(off):
                va = buf_a[pl.ds(off, L)]
                vb = buf_b[pl.ds(off, L)]
                buf_a[pl.ds(off, L)] = jnp.maximum(va - delta_a, zf) * rinv_a
                buf_b[pl.ds(off, L)] = jnp.maximum(vb - delta_b, zf) * rinv_b

        n_pairs = RPW // 2
        ld_a = pltpu.async_copy(in_hbm.at[base], buf_a, sem_a)
        ld_b = pltpu.async_copy(in_hbm.at[base + 1], buf_b, sem_b)
        for p in range(n_pairs):
            r = base + 2 * p
            ld_a.wait()
            ld_b.wait()
            process_pair()
            st_a = pltpu.async_copy(buf_a, out_hbm.at[r], sem_a)
            st_b = pltpu.async_copy(buf_b, out_hbm.at[r + 1], sem_b)
            if p + 1 < n_pairs:
                st_a.wait()
                ld_a = pltpu.async_copy(in_hbm.at[r + 2], buf_a, sem_a)
                st_b.wait()
                ld_b = pltpu.async_copy(in_hbm.at[r + 3], buf_b, sem_b)
        st_a.wait()
        st_b.wait()

    return sc_topk_norm


_sc_kernel = _build()


def kernel(attn_s):
    return _sc_kernel(attn_s)
